# baseline (speedup 1.0000x reference)
import jax
import jax.numpy as jnp
import numpy as np
from jax import lax
from jax.experimental import pallas as pl

B = 8
P = 1024
S = 512
K = 64
RSQ = np.float32(0.2 * 0.2)
RSQ_BITS = int(np.array(0.2 * 0.2, np.float32).view(np.int32))


def _fps_body(posT_ref, poss_ref):
    px = posT_ref[0]
    py = posT_ref[1]
    pz = posT_ref[2]
    iota = lax.broadcasted_iota(jnp.int32, (B, P), 1).astype(jnp.float32)
    li = lax.broadcasted_iota(jnp.int32, (B, 128), 1)

    def step(i, carry):
        dist, far = carry
        oh = iota == far
        cx = jnp.sum(jnp.where(oh, px, 0.0), axis=1, keepdims=True)
        cy = jnp.sum(jnp.where(oh, py, 0.0), axis=1, keepdims=True)
        cz = jnp.sum(jnp.where(oh, pz, 0.0), axis=1, keepdims=True)
        val = (jnp.where(li == 0, cx, 0.0) + jnp.where(li == 1, cy, 0.0)
               + jnp.where(li == 2, cz, 0.0))
        poss_ref[pl.ds(i, 1)] = val.reshape(1, B, 128)
        d = (px - cx) ** 2 + (py - cy) ** 2 + (pz - cz) ** 2
        dist = jnp.minimum(dist, d)
        mx = jnp.max(dist, axis=1, keepdims=True)
        far = jnp.min(jnp.where(dist == mx, iota, jnp.float32(P)), axis=1,
                      keepdims=True)
        return dist, far

    dist0 = jnp.full((B, P), jnp.inf, dtype=jnp.float32)
    far0 = jnp.zeros((B, 1), dtype=jnp.float32)
    lax.fori_loop(0, S, step, (dist0, far0))


def _fused_body(posT_ref, posb_ref, xb_ref, poss_ref, W1x_ref, W1p_ref,
                b1_ref, W2_ref, b2_ref, W3_ref, b3_ref, out_ref):
    px = posT_ref[0, 0:1, :]
    py = posT_ref[0, 1:2, :]
    pz = posT_ref[0, 2:3, :]
    sx = poss_ref[0, :, 0:1]
    sy = poss_ref[0, :, 1:2]
    sz = poss_ref[0, :, 2:3]
    d2 = (sx - px) ** 2 + (sy - py) ** 2 + (sz - pz) ** 2
    d2 = jnp.where(d2 <= RSQ, d2, jnp.inf)

    F = (jnp.dot(xb_ref[0], W1x_ref[...],
                 preferred_element_type=jnp.float32)
         + jnp.dot(posb_ref[0], W1p_ref[...],
                   preferred_element_type=jnp.float32))
    Gm = jnp.dot(poss_ref[0], W1p_ref[...],
                 preferred_element_type=jnp.float32) - b1_ref[...]
    Fb16 = F.astype(jnp.bfloat16)
    W2b16 = W2_ref[...].astype(jnp.bfloat16)
    W3b16 = W3_ref[...].astype(jnp.bfloat16)

    d2b = lax.bitcast_convert_type(d2, jnp.int32)
    total = jnp.sum(jnp.where(d2 < jnp.inf, 1.0, 0.0), axis=1,
                    keepdims=True)
    target = jnp.minimum(total, jnp.float32(K))

    def bs(i, carry):
        lo, hi = carry
        mid = (lo + hi) >> 1
        cnt = jnp.sum(jnp.where(d2b <= mid, 1.0, 0.0), axis=1,
                      keepdims=True)
        ge = cnt >= target
        return jnp.where(ge, lo, mid + 1), jnp.where(ge, mid, hi)

    lo0 = jnp.zeros((S, 1), dtype=jnp.int32)
    hi0 = jnp.full((S, 1), RSQ_BITS, dtype=jnp.int32)
    _, vstar = lax.fori_loop(0, 30, bs, (lo0, hi0))
    selb = jnp.where(d2b <= vstar, 1.0, 0.0)

    rio = lax.broadcasted_iota(jnp.int32, (P, P), 0)
    cio = lax.broadcasted_iota(jnp.int32, (P, P), 1)
    tri = jnp.where(rio <= cio, 1.0, 0.0)
    cumsel = jnp.dot(selb, tri, preferred_element_type=jnp.float32) * selb

    macc = jnp.full((S, 128), -jnp.inf, dtype=jnp.float32)
    for k in range(K):
        ohb = (cumsel == jnp.float32(k + 1)).astype(jnp.bfloat16)
        rows = jnp.dot(ohb, Fb16,
                       preferred_element_type=jnp.float32)
        h1 = jnp.maximum(rows - Gm, 0.0).astype(jnp.bfloat16)
        h2 = jnp.maximum(jnp.dot(h1, W2b16,
                                 preferred_element_type=jnp.float32)
                         + b2_ref[...], 0.0).astype(jnp.bfloat16)
        h3 = jnp.dot(h2, W3b16,
                     preferred_element_type=jnp.float32) + b3_ref[...]
        macc = jnp.maximum(macc, jnp.where(total > jnp.float32(k), h3,
                                           -jnp.inf))
    out_ref[...] = macc


def kernel(x, pos, batch, W1, b1, W2, b2, W3, b3):
    pos_b = pos.reshape(B, P, 3)
    x_b = x.reshape(B, P, 3)
    posT = pos_b.transpose(2, 0, 1)
    W1x = W1[0:3, :]
    W1p = W1[3:6, :]

    poss_raw = pl.pallas_call(
        _fps_body,
        out_shape=jax.ShapeDtypeStruct((S, B, 128), jnp.float32),
    )(posT)
    poss_b = poss_raw[:, :, 0:3].transpose(1, 0, 2)

    out_x = pl.pallas_call(
        _fused_body,
        grid=(B,),
        in_specs=[
            pl.BlockSpec((1, 3, P), lambda b: (b, 0, 0)),
            pl.BlockSpec((1, P, 3), lambda b: (b, 0, 0)),
            pl.BlockSpec((1, P, 3), lambda b: (b, 0, 0)),
            pl.BlockSpec((1, S, 3), lambda b: (b, 0, 0)),
            pl.BlockSpec((3, 64), lambda b: (0, 0)),
            pl.BlockSpec((3, 64), lambda b: (0, 0)),
            pl.BlockSpec((1, 64), lambda b: (0, 0)),
            pl.BlockSpec((64, 64), lambda b: (0, 0)),
            pl.BlockSpec((1, 64), lambda b: (0, 0)),
            pl.BlockSpec((64, 128), lambda b: (0, 0)),
            pl.BlockSpec((1, 128), lambda b: (0, 0)),
        ],
        out_specs=pl.BlockSpec((S, 128), lambda b: (b, 0)),
        out_shape=jax.ShapeDtypeStruct((B * S, 128), jnp.float32),
    )(pos_b.transpose(0, 2, 1), pos_b, x_b, poss_b, W1x, W1p,
      b1.reshape(1, 64), W2, b2.reshape(1, 64), W3, b3.reshape(1, 128))

    out_pos = poss_b.reshape(B * S, 3)
    out_batch = jnp.repeat(jnp.arange(B, dtype=jnp.int32), S)
    return (out_x, out_pos, out_batch)


# speedup vs baseline: 1.0364x; 1.0364x over previous
import jax
import jax.numpy as jnp
import numpy as np
from jax import lax
from jax.experimental import pallas as pl

B = 8
P = 1024
S = 512
K = 64
RSQ = np.float32(0.2 * 0.2)
RSQ_BITS = int(np.array(0.2 * 0.2, np.float32).view(np.int32))


def _fps_body(posT_ref, poss_ref):
    px = posT_ref[0]
    py = posT_ref[1]
    pz = posT_ref[2]
    iota = lax.broadcasted_iota(jnp.int32, (B, P), 1).astype(jnp.float32)
    li = lax.broadcasted_iota(jnp.int32, (B, 128), 1)

    def step(i, carry):
        dist, far = carry
        oh = iota == far
        cx = jnp.sum(jnp.where(oh, px, 0.0), axis=1, keepdims=True)
        cy = jnp.sum(jnp.where(oh, py, 0.0), axis=1, keepdims=True)
        cz = jnp.sum(jnp.where(oh, pz, 0.0), axis=1, keepdims=True)
        val = (jnp.where(li == 0, cx, 0.0) + jnp.where(li == 1, cy, 0.0)
               + jnp.where(li == 2, cz, 0.0))
        poss_ref[pl.ds(i, 1)] = val.reshape(1, B, 128)
        d = (px - cx) ** 2 + (py - cy) ** 2 + (pz - cz) ** 2
        dist = jnp.minimum(dist, d)
        mx = jnp.max(dist, axis=1, keepdims=True)
        far = jnp.min(jnp.where(dist == mx, iota, jnp.float32(P)), axis=1,
                      keepdims=True)
        return dist, far

    dist0 = jnp.full((B, P), jnp.inf, dtype=jnp.float32)
    far0 = jnp.zeros((B, 1), dtype=jnp.float32)
    lax.fori_loop(0, S, step, (dist0, far0))


def _fused_body(posT_ref, posb_ref, xb_ref, poss_ref, W1x_ref, W1p_ref,
                b1_ref, W2_ref, b2_ref, W3_ref, b3_ref, out_ref):
    px = posT_ref[0, 0:1, :]
    py = posT_ref[0, 1:2, :]
    pz = posT_ref[0, 2:3, :]
    sx = poss_ref[0, :, 0:1]
    sy = poss_ref[0, :, 1:2]
    sz = poss_ref[0, :, 2:3]
    d2 = (sx - px) ** 2 + (sy - py) ** 2 + (sz - pz) ** 2
    d2 = jnp.where(d2 <= RSQ, d2, jnp.inf)

    F = (jnp.dot(xb_ref[0], W1x_ref[...],
                 preferred_element_type=jnp.float32)
         + jnp.dot(posb_ref[0], W1p_ref[...],
                   preferred_element_type=jnp.float32))
    Gm = jnp.dot(poss_ref[0], W1p_ref[...],
                 preferred_element_type=jnp.float32) - b1_ref[...]
    Fb16 = F.astype(jnp.bfloat16)
    W2b16 = W2_ref[...].astype(jnp.bfloat16)
    W3b16 = W3_ref[...].astype(jnp.bfloat16)

    d2b = lax.bitcast_convert_type(d2, jnp.int32)
    total = jnp.sum(jnp.where(d2 < jnp.inf, 1.0, 0.0), axis=1,
                    keepdims=True)
    target = jnp.minimum(total, jnp.float32(K))

    def bs(i, carry):
        lo, hi = carry
        mid = (lo + hi) >> 1
        cnt = jnp.sum(jnp.where(d2b <= mid, 1.0, 0.0), axis=1,
                      keepdims=True)
        ge = cnt >= target
        return jnp.where(ge, lo, mid + 1), jnp.where(ge, mid, hi)

    lo0 = jnp.zeros((S, 1), dtype=jnp.int32)
    hi0 = jnp.full((S, 1), RSQ_BITS, dtype=jnp.int32)
    _, vstar = lax.fori_loop(0, 30, bs, (lo0, hi0))
    selb = jnp.where(d2b <= vstar, 1.0, 0.0)

    rio = lax.broadcasted_iota(jnp.int32, (P, P), 0)
    cio = lax.broadcasted_iota(jnp.int32, (P, P), 1)
    tri = jnp.where(rio <= cio, 1.0, 0.0)
    cumsel = jnp.dot(selb, tri, preferred_element_type=jnp.float32) * selb
    cumselb = cumsel.astype(jnp.bfloat16)

    macc = jnp.full((S, 128), -jnp.inf, dtype=jnp.float32)
    for k in range(K):
        ohb = (cumselb == jnp.bfloat16(k + 1)).astype(jnp.bfloat16)
        rows = jnp.dot(ohb, Fb16,
                       preferred_element_type=jnp.float32)
        h1 = jnp.maximum(rows - Gm, 0.0).astype(jnp.bfloat16)
        h2 = jnp.maximum(jnp.dot(h1, W2b16,
                                 preferred_element_type=jnp.float32)
                         + b2_ref[...], 0.0).astype(jnp.bfloat16)
        h3 = jnp.dot(h2, W3b16,
                     preferred_element_type=jnp.float32) + b3_ref[...]
        macc = jnp.maximum(macc, jnp.where(total > jnp.float32(k), h3,
                                           -jnp.inf))
    out_ref[...] = macc


def kernel(x, pos, batch, W1, b1, W2, b2, W3, b3):
    pos_b = pos.reshape(B, P, 3)
    x_b = x.reshape(B, P, 3)
    posT = pos_b.transpose(2, 0, 1)
    W1x = W1[0:3, :]
    W1p = W1[3:6, :]

    poss_raw = pl.pallas_call(
        _fps_body,
        out_shape=jax.ShapeDtypeStruct((S, B, 128), jnp.float32),
    )(posT)
    poss_b = poss_raw[:, :, 0:3].transpose(1, 0, 2)

    out_x = pl.pallas_call(
        _fused_body,
        grid=(B,),
        in_specs=[
            pl.BlockSpec((1, 3, P), lambda b: (b, 0, 0)),
            pl.BlockSpec((1, P, 3), lambda b: (b, 0, 0)),
            pl.BlockSpec((1, P, 3), lambda b: (b, 0, 0)),
            pl.BlockSpec((1, S, 3), lambda b: (b, 0, 0)),
            pl.BlockSpec((3, 64), lambda b: (0, 0)),
            pl.BlockSpec((3, 64), lambda b: (0, 0)),
            pl.BlockSpec((1, 64), lambda b: (0, 0)),
            pl.BlockSpec((64, 64), lambda b: (0, 0)),
            pl.BlockSpec((1, 64), lambda b: (0, 0)),
            pl.BlockSpec((64, 128), lambda b: (0, 0)),
            pl.BlockSpec((1, 128), lambda b: (0, 0)),
        ],
        out_specs=pl.BlockSpec((S, 128), lambda b: (b, 0)),
        out_shape=jax.ShapeDtypeStruct((B * S, 128), jnp.float32),
    )(pos_b.transpose(0, 2, 1), pos_b, x_b, poss_b, W1x, W1p,
      b1.reshape(1, 64), W2, b2.reshape(1, 64), W3, b3.reshape(1, 128))

    out_pos = poss_b.reshape(B * S, 3)
    out_batch = jnp.repeat(jnp.arange(B, dtype=jnp.int32), S)
    return (out_x, out_pos, out_batch)


# speedup vs baseline: 1.1351x; 1.0953x over previous
import jax
import jax.numpy as jnp
import numpy as np
from jax import lax
from jax.experimental import pallas as pl

B = 8
P = 1024
S = 512
K = 64
RSQ = np.float32(0.2 * 0.2)
RSQ_BITS = int(np.array(0.2 * 0.2, np.float32).view(np.int32))


def _fps_body(posT_ref, poss_ref):
    px = posT_ref[0]
    py = posT_ref[1]
    pz = posT_ref[2]
    iota = lax.broadcasted_iota(jnp.int32, (B, P), 1).astype(jnp.float32)
    li = lax.broadcasted_iota(jnp.int32, (B, 128), 1)

    def step(i, carry):
        dist, far = carry
        oh = iota == far
        cx = jnp.sum(jnp.where(oh, px, 0.0), axis=1, keepdims=True)
        cy = jnp.sum(jnp.where(oh, py, 0.0), axis=1, keepdims=True)
        cz = jnp.sum(jnp.where(oh, pz, 0.0), axis=1, keepdims=True)
        val = (jnp.where(li == 0, cx, 0.0) + jnp.where(li == 1, cy, 0.0)
               + jnp.where(li == 2, cz, 0.0))
        poss_ref[pl.ds(i, 1)] = val.reshape(1, B, 128)
        d = (px - cx) ** 2 + (py - cy) ** 2 + (pz - cz) ** 2
        dist = jnp.minimum(dist, d)
        mx = jnp.max(dist, axis=1, keepdims=True)
        far = jnp.min(jnp.where(dist == mx, iota, jnp.float32(P)), axis=1,
                      keepdims=True)
        return dist, far

    dist0 = jnp.full((B, P), jnp.inf, dtype=jnp.float32)
    far0 = jnp.zeros((B, 1), dtype=jnp.float32)
    lax.fori_loop(0, S, step, (dist0, far0))


def _fused_body(posb_ref, xbT_ref, posbT_ref, possT_ref, W1xT_ref, W1pT_ref,
                b1c_ref, W2T_ref, b2c_ref, W3T_ref, b3c_ref, out_ref):
    px = posb_ref[0][:, 0:1]
    py = posb_ref[0][:, 1:2]
    pz = posb_ref[0][:, 2:3]
    sx = possT_ref[0, 0:1, :]
    sy = possT_ref[0, 1:2, :]
    sz = possT_ref[0, 2:3, :]
    d2 = (px - sx) ** 2 + (py - sy) ** 2 + (pz - sz) ** 2
    d2 = jnp.where(d2 <= RSQ, d2, jnp.inf)

    FT = (jnp.dot(W1xT_ref[...], xbT_ref[0],
                  preferred_element_type=jnp.float32)
          + jnp.dot(W1pT_ref[...], posbT_ref[0],
                    preferred_element_type=jnp.float32))
    GmT = jnp.dot(W1pT_ref[...], possT_ref[0],
                  preferred_element_type=jnp.float32) - b1c_ref[...]
    FTb = FT.astype(jnp.bfloat16)
    W2Tb = W2T_ref[...].astype(jnp.bfloat16)
    W3Tb = W3T_ref[...].astype(jnp.bfloat16)

    d2b = lax.bitcast_convert_type(d2, jnp.int32)
    total = jnp.sum(jnp.where(d2 < jnp.inf, 1.0, 0.0), axis=0,
                    keepdims=True)
    target = jnp.minimum(total, jnp.float32(K))

    def bs(i, carry):
        lo, hi = carry
        mid = (lo + hi) >> 1
        cnt = jnp.sum(jnp.where(d2b <= mid, 1.0, 0.0), axis=0,
                      keepdims=True)
        ge = cnt >= target
        return jnp.where(ge, lo, mid + 1), jnp.where(ge, mid, hi)

    lo0 = jnp.zeros((1, S), dtype=jnp.int32)
    hi0 = jnp.full((1, S), RSQ_BITS, dtype=jnp.int32)
    _, vstar = lax.fori_loop(0, 30, bs, (lo0, hi0))
    selb = jnp.where(d2b <= vstar, 1.0, 0.0)

    rio = lax.broadcasted_iota(jnp.int32, (P, P), 0)
    cio = lax.broadcasted_iota(jnp.int32, (P, P), 1)
    tri = jnp.where(rio >= cio, 1.0, 0.0)
    cum = jnp.dot(tri, selb, preferred_element_type=jnp.float32) * selb
    cumb = cum.astype(jnp.bfloat16)

    macc = jnp.full((128, S), -jnp.inf, dtype=jnp.float32)
    for k in range(K):
        ohb = (cumb == jnp.bfloat16(k + 1)).astype(jnp.bfloat16)
        rows = jnp.dot(FTb, ohb,
                       preferred_element_type=jnp.float32)
        h1 = jnp.maximum(rows - GmT, 0.0).astype(jnp.bfloat16)
        h2 = jnp.maximum(jnp.dot(W2Tb, h1,
                                 preferred_element_type=jnp.float32)
                         + b2c_ref[...], 0.0).astype(jnp.bfloat16)
        h3 = jnp.dot(W3Tb, h2,
                     preferred_element_type=jnp.float32) + b3c_ref[...]
        macc = jnp.maximum(macc, jnp.where(total > jnp.float32(k), h3,
                                           -jnp.inf))
    out_ref[0] = macc


def kernel(x, pos, batch, W1, b1, W2, b2, W3, b3):
    pos_b = pos.reshape(B, P, 3)
    x_b = x.reshape(B, P, 3)
    posT = pos_b.transpose(2, 0, 1)

    poss_raw = pl.pallas_call(
        _fps_body,
        out_shape=jax.ShapeDtypeStruct((S, B, 128), jnp.float32),
    )(posT)
    poss_b = poss_raw[:, :, 0:3].transpose(1, 0, 2)

    outT = pl.pallas_call(
        _fused_body,
        grid=(B,),
        in_specs=[
            pl.BlockSpec((1, P, 3), lambda b: (b, 0, 0)),
            pl.BlockSpec((1, 3, P), lambda b: (b, 0, 0)),
            pl.BlockSpec((1, 3, P), lambda b: (b, 0, 0)),
            pl.BlockSpec((1, 3, S), lambda b: (b, 0, 0)),
            pl.BlockSpec((64, 3), lambda b: (0, 0)),
            pl.BlockSpec((64, 3), lambda b: (0, 0)),
            pl.BlockSpec((64, 1), lambda b: (0, 0)),
            pl.BlockSpec((64, 64), lambda b: (0, 0)),
            pl.BlockSpec((64, 1), lambda b: (0, 0)),
            pl.BlockSpec((128, 64), lambda b: (0, 0)),
            pl.BlockSpec((128, 1), lambda b: (0, 0)),
        ],
        out_specs=pl.BlockSpec((1, 128, S), lambda b: (b, 0, 0)),
        out_shape=jax.ShapeDtypeStruct((B, 128, S), jnp.float32),
    )(pos_b, x_b.transpose(0, 2, 1), pos_b.transpose(0, 2, 1),
      poss_b.transpose(0, 2, 1), W1[0:3, :].T, W1[3:6, :].T,
      b1.reshape(64, 1), W2.T, b2.reshape(64, 1), W3.T, b3.reshape(128, 1))

    out_x = outT.transpose(0, 2, 1).reshape(B * S, 128)
    out_pos = poss_b.reshape(B * S, 3)
    out_batch = jnp.repeat(jnp.arange(B, dtype=jnp.int32), S)
    return (out_x, out_pos, out_batch)


# speedup vs baseline: 1.6685x; 1.4699x over previous
import jax
import jax.numpy as jnp
import numpy as np
from jax import lax
from jax.experimental import pallas as pl

B = 8
P = 1024
S = 512
K = 64
RSQ = np.float32(0.2 * 0.2)
RSQ_BITS = int(np.array(0.2 * 0.2, np.float32).view(np.int32))


def _fps_body(posT_ref, poss_ref):
    px = posT_ref[0]
    py = posT_ref[1]
    pz = posT_ref[2]
    iota = lax.broadcasted_iota(jnp.int32, (B, P), 1).astype(jnp.float32)
    li = lax.broadcasted_iota(jnp.int32, (B, 128), 1)

    def step(i, carry):
        dist, far = carry
        oh = iota == far
        cx = jnp.sum(jnp.where(oh, px, 0.0), axis=1, keepdims=True)
        cy = jnp.sum(jnp.where(oh, py, 0.0), axis=1, keepdims=True)
        cz = jnp.sum(jnp.where(oh, pz, 0.0), axis=1, keepdims=True)
        val = (jnp.where(li == 0, cx, 0.0) + jnp.where(li == 1, cy, 0.0)
               + jnp.where(li == 2, cz, 0.0))
        poss_ref[pl.ds(i, 1)] = val.reshape(1, B, 128)
        d = (px - cx) ** 2 + (py - cy) ** 2 + (pz - cz) ** 2
        dist = jnp.minimum(dist, d)
        mx = jnp.max(dist, axis=1, keepdims=True)
        far = jnp.min(jnp.where(dist == mx, iota, jnp.float32(P)), axis=1,
                      keepdims=True)
        return dist, far

    dist0 = jnp.full((B, P), jnp.inf, dtype=jnp.float32)
    far0 = jnp.zeros((B, 1), dtype=jnp.float32)
    lax.fori_loop(0, S, step, (dist0, far0))


def _fused_body(posb_ref, xbT_ref, posbT_ref, possT_ref, W1xT_ref, W1pT_ref,
                b1c_ref, W2T_ref, b2c_ref, W3T_ref, b3c_ref, out_ref):
    px = posb_ref[0][:, 0:1]
    py = posb_ref[0][:, 1:2]
    pz = posb_ref[0][:, 2:3]
    sx = possT_ref[0, 0:1, :]
    sy = possT_ref[0, 1:2, :]
    sz = possT_ref[0, 2:3, :]
    d2 = (px - sx) ** 2 + (py - sy) ** 2 + (pz - sz) ** 2
    d2 = jnp.where(d2 <= RSQ, d2, jnp.inf)

    FT = (jnp.dot(W1xT_ref[...], xbT_ref[0],
                  preferred_element_type=jnp.float32)
          + jnp.dot(W1pT_ref[...], posbT_ref[0],
                    preferred_element_type=jnp.float32))
    GmT = jnp.dot(W1pT_ref[...], possT_ref[0],
                  preferred_element_type=jnp.float32) - b1c_ref[...]
    FTb = FT.astype(jnp.bfloat16)
    W2Tb = W2T_ref[...].astype(jnp.bfloat16)
    W3Tb = W3T_ref[...].astype(jnp.bfloat16)

    d2b = lax.bitcast_convert_type(d2, jnp.int32)
    total = jnp.sum(jnp.where(d2 < jnp.inf, 1.0, 0.0), axis=0,
                    keepdims=True)
    target = jnp.minimum(total, jnp.float32(K))

    def bs(i, carry):
        lo, hi = carry
        mid = (lo + hi) >> 1
        cnt = jnp.sum(jnp.where(d2b <= mid, 1.0, 0.0), axis=0,
                      keepdims=True)
        ge = cnt >= target
        return jnp.where(ge, lo, mid + 1), jnp.where(ge, mid, hi)

    lo0 = jnp.zeros((1, S), dtype=jnp.int32)
    hi0 = jnp.full((1, S), RSQ_BITS, dtype=jnp.int32)
    _, vstar = lax.fori_loop(0, 30, bs, (lo0, hi0))
    selb = jnp.where(d2b <= vstar, 1.0, 0.0)

    rio = lax.broadcasted_iota(jnp.int32, (P, P), 0)
    cio = lax.broadcasted_iota(jnp.int32, (P, P), 1)
    tri = jnp.where(rio >= cio, 1.0, 0.0)
    cum = jnp.dot(tri, selb, preferred_element_type=jnp.float32) * selb
    cumb = cum.astype(jnp.bfloat16)

    macc = jnp.full((128, S), -jnp.inf, dtype=jnp.float32)
    for k in range(K):
        ohb = jnp.where(cumb == jnp.bfloat16(k + 1), jnp.bfloat16(1),
                        jnp.bfloat16(0))
        rows = jnp.dot(FTb, ohb,
                       preferred_element_type=jnp.float32)
        h1 = jnp.maximum(rows - GmT, 0.0).astype(jnp.bfloat16)
        h2 = jnp.maximum(jnp.dot(W2Tb, h1,
                                 preferred_element_type=jnp.float32)
                         + b2c_ref[...], 0.0).astype(jnp.bfloat16)
        h3 = jnp.dot(W3Tb, h2,
                     preferred_element_type=jnp.float32) + b3c_ref[...]
        macc = jnp.maximum(macc, jnp.where(total > jnp.float32(k), h3,
                                           -jnp.inf))
    out_ref[0] = macc


def kernel(x, pos, batch, W1, b1, W2, b2, W3, b3):
    pos_b = pos.reshape(B, P, 3)
    x_b = x.reshape(B, P, 3)
    posT = pos_b.transpose(2, 0, 1)

    poss_raw = pl.pallas_call(
        _fps_body,
        out_shape=jax.ShapeDtypeStruct((S, B, 128), jnp.float32),
    )(posT)
    poss_b = poss_raw[:, :, 0:3].transpose(1, 0, 2)

    outT = pl.pallas_call(
        _fused_body,
        grid=(B,),
        in_specs=[
            pl.BlockSpec((1, P, 3), lambda b: (b, 0, 0)),
            pl.BlockSpec((1, 3, P), lambda b: (b, 0, 0)),
            pl.BlockSpec((1, 3, P), lambda b: (b, 0, 0)),
            pl.BlockSpec((1, 3, S), lambda b: (b, 0, 0)),
            pl.BlockSpec((64, 3), lambda b: (0, 0)),
            pl.BlockSpec((64, 3), lambda b: (0, 0)),
            pl.BlockSpec((64, 1), lambda b: (0, 0)),
            pl.BlockSpec((64, 64), lambda b: (0, 0)),
            pl.BlockSpec((64, 1), lambda b: (0, 0)),
            pl.BlockSpec((128, 64), lambda b: (0, 0)),
            pl.BlockSpec((128, 1), lambda b: (0, 0)),
        ],
        out_specs=pl.BlockSpec((1, 128, S), lambda b: (b, 0, 0)),
        out_shape=jax.ShapeDtypeStruct((B, 128, S), jnp.float32),
    )(pos_b, x_b.transpose(0, 2, 1), pos_b.transpose(0, 2, 1),
      poss_b.transpose(0, 2, 1), W1[0:3, :].T, W1[3:6, :].T,
      b1.reshape(64, 1), W2.T, b2.reshape(64, 1), W3.T, b3.reshape(128, 1))

    out_x = outT.transpose(0, 2, 1).reshape(B * S, 128)
    out_pos = poss_b.reshape(B * S, 3)
    out_batch = jnp.repeat(jnp.arange(B, dtype=jnp.int32), S)
    return (out_x, out_pos, out_batch)
